# baseline (speedup 1.0000x reference)
import functools

import jax
import jax.numpy as jnp
from jax import lax
from jax.experimental import pallas as pl
from jax.experimental.pallas import tpu as pltpu
from jax.experimental.pallas import tpu_sc as plsc

_K = 5
_LANES = 16
_NEG = -3.0e38


def _insert_topk(t, ts):
    out = []
    for cur in ts:
        hi = jnp.maximum(cur, t)
        t = jnp.minimum(cur, t)
        out.append(hi)
    return out


def _sc_topk_body(x_hbm, out_hbm, buf0, buf1, outv, sem0, sem1,
                  *, rows_per_worker, rb, s, unroll):
    wid = lax.axis_index("s") * 2 + lax.axis_index("c")
    row0 = wid * rows_per_worker
    nblk = rows_per_worker // rb
    iota = lax.iota(jnp.int32, _LANES)

    def start(blk, buf, sem):
        off = (row0 + blk * rb) * s
        return pltpu.async_copy(x_hbm.at[pl.ds(off, rb * s)], buf, sem)

    def wait(buf, sem):
        pltpu.make_async_copy(x_hbm.at[pl.ds(0, rb * s)], buf, sem).wait()

    def process(blk, buf):
        def g_body(g, _):
            rowbase = (g * _LANES + iota) * s
            end = rowbase + s
            init = tuple(jnp.full((_LANES,), _NEG, jnp.float32)
                         for _ in range(_K)) + (rowbase + iota,)

            def step(t, carry):
                ts, idx = list(carry[:_K]), carry[_K]
                v = plsc.load_gather(buf, [idx])
                ts = _insert_topk(v, ts)
                nxt = idx + 1
                nxt = jnp.where(nxt == end, rowbase, nxt)
                return tuple(ts) + (nxt,)

            carry = lax.fori_loop(0, s, step, init, unroll=unroll)
            res = carry[0]
            for t_ in carry[1:_K]:
                res = res + t_
            outv[pl.ds(blk * rb + g * _LANES, _LANES)] = res
            return 0

        lax.fori_loop(0, rb // _LANES, g_body, 0, unroll=False)

    start(0, buf0, sem0)

    def pair_body(i, _):
        b0 = 2 * i
        start(b0 + 1, buf1, sem1)
        wait(buf0, sem0)
        process(b0, buf0)

        @pl.when(i < nblk // 2 - 1)
        def _():
            start(b0 + 2, buf0, sem0)

        wait(buf1, sem1)
        process(b0 + 1, buf1)
        return 0

    lax.fori_loop(0, nblk // 2, pair_body, 0, unroll=False)
    pltpu.sync_copy(outv, out_hbm.at[pl.ds(row0, rows_per_worker)])


def _matmul_body(sim_ref, w_ref, o_ref):
    o_ref[...] = jax.lax.dot_general(
        sim_ref[...], w_ref[...], (((1,), (0,)), ((), ())),
        precision=jax.lax.Precision.HIGHEST,
        preferred_element_type=jnp.float32) * (1.0 / _K)


def kernel(prototype_activations, W):
    b, p = prototype_activations.shape[:2]
    s = prototype_activations.shape[2] * prototype_activations.shape[3]
    rows = b * p
    n_workers = 32
    rows_per_worker = rows // n_workers
    rb = 32

    x1 = prototype_activations.reshape(rows * s)

    body = functools.partial(
        _sc_topk_body, rows_per_worker=rows_per_worker, rb=rb, s=s, unroll=8)
    sc_call = pl.kernel(
        body,
        mesh=plsc.VectorSubcoreMesh(core_axis_name="c", subcore_axis_name="s"),
        compiler_params=pltpu.CompilerParams(needs_layout_passes=False),
        out_type=jax.ShapeDtypeStruct((rows,), jnp.float32),
        scratch_types=[
            pltpu.VMEM((rb * s,), jnp.float32),
            pltpu.VMEM((rb * s,), jnp.float32),
            pltpu.VMEM((rows_per_worker,), jnp.float32),
            pltpu.SemaphoreType.DMA,
            pltpu.SemaphoreType.DMA,
        ],
    )
    top5_sums = sc_call(x1)
    sim = top5_sums.reshape(b, p)

    c = W.shape[0]
    logits = pl.pallas_call(
        _matmul_body,
        in_specs=[
            pl.BlockSpec((b, p), lambda: (0, 0)),
            pl.BlockSpec((p, c), lambda: (0, 0)),
        ],
        out_specs=pl.BlockSpec((b, c), lambda: (0, 0)),
        out_shape=jax.ShapeDtypeStruct((b, c), jnp.float32),
    )(sim, W.T)
    return logits


# speedup vs baseline: 2.3082x; 2.3082x over previous
import jax
import jax.numpy as jnp
from jax.experimental import pallas as pl
from jax.experimental.pallas import tpu as pltpu

_K = 5
_NEG = -3.0e38


def _topk_native_kernel(x_ref, w_ref, o_ref, *ts_refs):
    i = pl.program_id(0)
    j = pl.program_id(1)
    nj = pl.num_programs(1)
    x = x_ref[0, 0]

    @pl.when(jnp.logical_and(i == 0, j == 0))
    def _():
        for r in ts_refs:
            r[...] = jnp.full(r.shape, _NEG, jnp.float32)

    t = x
    for r in ts_refs:
        cur = r[...]
        hi = jnp.maximum(cur, t)
        t = jnp.minimum(cur, t)
        r[...] = hi

    @pl.when(jnp.logical_and(i == pl.num_programs(0) - 1, j == nj - 1))
    def _():
        acc = ts_refs[0][...]
        for r in ts_refs[1:]:
            acc = acc + r[...]
        sim = acc * (1.0 / _K)
        o_ref[...] = jax.lax.dot_general(
            sim, w_ref[...], (((0,), (1,)), ((), ())),
            precision=jax.lax.Precision.HIGHEST,
            preferred_element_type=jnp.float32)


def kernel(prototype_activations, W):
    b, p, h, w = prototype_activations.shape
    c = W.shape[0]
    xt = jnp.transpose(prototype_activations, (2, 3, 1, 0))

    out = pl.pallas_call(
        _topk_native_kernel,
        grid=(h, w),
        in_specs=[
            pl.BlockSpec((1, 1, p, b), lambda i, j: (i, j, 0, 0)),
            pl.BlockSpec((c, p), lambda i, j: (0, 0)),
        ],
        out_specs=pl.BlockSpec((b, c), lambda i, j: (0, 0)),
        out_shape=jax.ShapeDtypeStruct((b, c), jnp.float32),
        scratch_shapes=[pltpu.VMEM((p, b), jnp.float32) for _ in range(_K)],
    )(xt, W)
    return out


# speedup vs baseline: 12.3968x; 5.3707x over previous
import jax
import jax.numpy as jnp
from jax.experimental import pallas as pl
from jax.experimental.pallas import tpu as pltpu

_K = 5
_NEG = -3.0e38


def _topk_native_kernel(x_ref, w_ref, o_ref, *ts_refs):
    i = pl.program_id(0)
    nw = x_ref.shape[1]

    @pl.when(i == 0)
    def _():
        neg = jnp.full(ts_refs[0].shape, _NEG, jnp.float32)
        for r in ts_refs:
            r[...] = neg

    ts = [r[...] for r in ts_refs]

    for k in range(nw):
        t = x_ref[0, k]
        for r_i in range(_K):
            cur = ts[r_i]
            hi = jnp.maximum(cur, t)
            t = jnp.minimum(cur, t)
            ts[r_i] = hi

    for r, t in zip(ts_refs, ts):
        r[...] = t

    @pl.when(i == pl.num_programs(0) - 1)
    def _():
        acc = ts[0]
        for t in ts[1:]:
            acc = acc + t
        sim = acc * (1.0 / _K)
        o_ref[...] = jax.lax.dot_general(
            sim, w_ref[...], (((0,), (1,)), ((), ())),
            precision=jax.lax.Precision.HIGHEST,
            preferred_element_type=jnp.float32)


def kernel(prototype_activations, W):
    b, p, h, w = prototype_activations.shape
    c = W.shape[0]
    xt = jnp.transpose(prototype_activations, (2, 3, 1, 0))

    out = pl.pallas_call(
        _topk_native_kernel,
        grid=(h,),
        in_specs=[
            pl.BlockSpec((1, w, p, b), lambda i: (i, 0, 0, 0)),
            pl.BlockSpec((c, p), lambda i: (0, 0)),
        ],
        out_specs=pl.BlockSpec((b, c), lambda i: (0, 0)),
        out_shape=jax.ShapeDtypeStruct((b, c), jnp.float32),
        scratch_shapes=[pltpu.VMEM((p, b), jnp.float32) for _ in range(_K)],
    )(xt, W)
    return out
